# Initial kernel scaffold; baseline (speedup 1.0000x reference)
import jax
import jax.numpy as jnp
from jax.experimental import pallas as pl


def kernel(x, edge_index, W1, b1, W2, b2):
    raise NotImplementedError("write your pallas kernel here")


# speedup vs baseline: 20.0331x; 20.0331x over previous
import dataclasses
import functools

import jax
import jax.numpy as jnp
from jax import lax
from jax.experimental import pallas as pl
from jax.experimental.pallas import tpu as pltpu
from jax.experimental.pallas import tpu_sc as plsc

N = 10000
F = 128
C = 16
E = 320000
NC = 2
NS = 16
NW = NC * NS
EPT = E // NW
W = 80
NCHUNK = EPT // W
B0 = 632
BL = N - (NS - 1) * B0

_mesh = plsc.VectorSubcoreMesh(core_axis_name="c", subcore_axis_name="s")

_cp = pltpu.CompilerParams()
if "needs_layout_passes" in pltpu.CompilerParams.__dataclass_fields__:
    _cp = dataclasses.replace(_cp, needs_layout_passes=False)


def _part_init(zeros_hbm, acc_sh, s):
    base = pl.multiple_of(s * B0, 8)

    @pl.when(s < NS - 1)
    def _():
        pltpu.sync_copy(zeros_hbm, acc_sh.at[pl.ds(base, B0)])

    @pl.when(s == NS - 1)
    def _():
        pltpu.sync_copy(zeros_hbm.at[pl.ds(0, BL)], acc_sh.at[pl.ds(base, BL)])


def _part_copyout(acc_sh, out_hbm, c, s):
    base = pl.multiple_of(s * B0, 8)

    @pl.when(s < NS - 1)
    def _():
        pltpu.sync_copy(acc_sh.at[pl.ds(base, B0)],
                        out_hbm.at[c, pl.ds(base, B0)])

    @pl.when(s == NS - 1)
    def _():
        pltpu.sync_copy(acc_sh.at[pl.ds(base, BL)],
                        out_hbm.at[c, pl.ds(base, BL)])


@functools.partial(
    pl.kernel, mesh=_mesh,
    out_type=jax.ShapeDtypeStruct((NW, N), jnp.float32),
    compiler_params=_cp,
    scratch_types=[
        pltpu.VMEM((NCHUNK, W), jnp.int32),
        pltpu.VMEM((N,), jnp.float32),
    ],
)
def _sc_degree(dst_hbm, out_hbm, dst_v, hist_v):
    c = lax.axis_index("c")
    s = lax.axis_index("s")
    wid = s * NC + c
    pltpu.sync_copy(dst_hbm.at[wid], dst_v)

    @pl.loop(0, N // 16)
    def _(i):
        hist_v[pl.ds(i * 16, 16)] = jnp.zeros((16,), jnp.float32)

    ones16 = jnp.ones((16,), jnp.float32)

    @pl.loop(0, NCHUNK)
    def _(i):
        for j in range(W // 16):
            idx16 = dst_v[i, pl.ds(j * 16, 16)]
            plsc.addupdate_scatter(hist_v, [idx16], ones16)

    pltpu.sync_copy(hist_v, out_hbm.at[wid])


def _make_sc_propagate(width):
    @functools.partial(
        pl.kernel, mesh=_mesh,
        out_type=jax.ShapeDtypeStruct((NC, N, width), jnp.float32),
        scratch_types=[
            pltpu.VMEM((NCHUNK, W), jnp.int32),
            pltpu.VMEM((NCHUNK, W), jnp.int32),
            pltpu.VMEM((W, width), jnp.float32),
            pltpu.VMEM_SHARED((N, width), jnp.float32),
        ],
    )
    def _sc_prop(src_hbm, dst_hbm, vals_hbm, zeros_hbm, out_hbm,
                 src_v, dst_v, rows_v, acc_sh):
        c = lax.axis_index("c")
        s = lax.axis_index("s")
        wid = s * NC + c
        pltpu.sync_copy(src_hbm.at[wid], src_v)
        pltpu.sync_copy(dst_hbm.at[wid], dst_v)
        _part_init(zeros_hbm, acc_sh, s)
        plsc.subcore_barrier()

        @pl.loop(0, NCHUNK)
        def _(i):
            pltpu.sync_copy(vals_hbm.at[src_v.at[i]], rows_v)
            pltpu.sync_copy(rows_v, acc_sh.at[dst_v.at[i]], add=True)

        plsc.subcore_barrier()
        _part_copyout(acc_sh, out_hbm, c, s)

    return _sc_prop


_sc_prop_f = _make_sc_propagate(F)


def _tc1_body(degp_ref, x_ref, w1_ref, xw1p_ref, dis_ref):
    deg = jnp.sum(degp_ref[...], axis=0)[:, None] + 1.0
    dis = 1.0 / jnp.sqrt(deg)
    xw1 = jnp.dot(x_ref[...], w1_ref[...], preferred_element_type=jnp.float32)
    xw1p_ref[...] = dis * xw1
    dis_ref[...] = dis


def _tc2_body(acc_ref, xw1p_ref, dis_ref, b1_ref, hp_ref):
    t = dis_ref[...] * (acc_ref[0] + acc_ref[1] + xw1p_ref[...]) + b1_ref[...]
    hp_ref[...] = dis_ref[...] * jnp.maximum(t, 0.0)


def _tc3_body(acc_ref, hp_ref, dis_ref, w2_ref, b2_ref, out_ref):
    g = dis_ref[...] * (acc_ref[0] + acc_ref[1] + hp_ref[...])
    out_ref[...] = (jnp.dot(g, w2_ref[...],
                            preferred_element_type=jnp.float32) + b2_ref[...])


def kernel(x, edge_index, W1, b1, W2, b2):
    x = x.astype(jnp.float32)
    ei = edge_index.astype(jnp.int32)
    src3 = ei[0].reshape(NW, NCHUNK, W)
    dst3 = ei[1].reshape(NW, NCHUNK, W)
    zeros_f = jnp.zeros((B0, F), jnp.float32)

    degp = _sc_degree(dst3)

    xw1p, dis = pl.pallas_call(
        _tc1_body,
        out_shape=[
            jax.ShapeDtypeStruct((N, F), jnp.float32),
            jax.ShapeDtypeStruct((N, 1), jnp.float32),
        ],
    )(degp, x, W1)

    acc1 = _sc_prop_f(src3, dst3, xw1p, zeros_f)

    hp = pl.pallas_call(
        _tc2_body,
        out_shape=jax.ShapeDtypeStruct((N, F), jnp.float32),
    )(acc1, xw1p, dis, b1)

    acc2 = _sc_prop_f(src3, dst3, hp, zeros_f)

    out = pl.pallas_call(
        _tc3_body,
        out_shape=jax.ShapeDtypeStruct((N, C), jnp.float32),
    )(acc2, hp, dis, W2, b2)

    return out


# speedup vs baseline: 24.9583x; 1.2459x over previous
import dataclasses
import functools

import jax
import jax.numpy as jnp
from jax import lax
from jax.experimental import pallas as pl
from jax.experimental.pallas import tpu as pltpu
from jax.experimental.pallas import tpu_sc as plsc

N = 10000
F = 128
C = 16
E = 320000
NC = 2
NS = 16
NW = NC * NS
W = 80
NCHUNK = 128
EPTP = NCHUNK * W
PADR = 64
NP = N + PADR
PAD_E = NW * EPTP - E

B0 = 632
BL = N - (NS - 1) * B0

_mesh = plsc.VectorSubcoreMesh(core_axis_name="c", subcore_axis_name="s")

_cp = pltpu.CompilerParams()
if "needs_layout_passes" in pltpu.CompilerParams.__dataclass_fields__:
    _cp = dataclasses.replace(_cp, needs_layout_passes=False)


def _part_init(zeros_hbm, acc_sh, s):
    base = pl.multiple_of(s * B0, 8)

    @pl.when(s < NS - 1)
    def _():
        pltpu.sync_copy(zeros_hbm, acc_sh.at[pl.ds(base, B0)])

    @pl.when(s == NS - 1)
    def _():
        pltpu.sync_copy(zeros_hbm.at[pl.ds(0, BL)], acc_sh.at[pl.ds(base, BL)])


def _part_copyout(acc_sh, out_hbm, c, s):
    base = pl.multiple_of(s * B0, 8)

    @pl.when(s < NS - 1)
    def _():
        pltpu.sync_copy(acc_sh.at[pl.ds(base, B0)],
                        out_hbm.at[c, pl.ds(base, B0)])

    @pl.when(s == NS - 1)
    def _():
        pltpu.sync_copy(acc_sh.at[pl.ds(base, BL)],
                        out_hbm.at[c, pl.ds(base, BL)])


@functools.partial(
    pl.kernel, mesh=_mesh,
    out_type=jax.ShapeDtypeStruct((NW, NP), jnp.float32),
    compiler_params=_cp,
    scratch_types=[
        pltpu.VMEM((NCHUNK, W), jnp.int32),
        pltpu.VMEM((NP,), jnp.float32),
    ],
)
def _sc_degree(dst_hbm, out_hbm, dst_v, hist_v):
    c = lax.axis_index("c")
    s = lax.axis_index("s")
    wid = s * NC + c
    pltpu.sync_copy(dst_hbm.at[wid], dst_v)

    @pl.loop(0, NP // 16)
    def _(i):
        hist_v[pl.ds(i * 16, 16)] = jnp.zeros((16,), jnp.float32)

    ones16 = jnp.ones((16,), jnp.float32)

    @pl.loop(0, NCHUNK)
    def _(i):
        for j in range(W // 16):
            idx16 = dst_v[i, pl.ds(j * 16, 16)]
            plsc.addupdate_scatter(hist_v, [idx16], ones16)

    pltpu.sync_copy(hist_v, out_hbm.at[wid])


@functools.partial(
    pl.kernel, mesh=_mesh,
    out_type=jax.ShapeDtypeStruct((NC, N, F), jnp.float32),
    scratch_types=[
        pltpu.VMEM((EPTP,), jnp.int32),
        pltpu.VMEM((NCHUNK, W), jnp.int32),
        pltpu.VMEM((2 * W, F), jnp.float32),
        pltpu.VMEM_SHARED((NP, F), jnp.float32),
        pltpu.SemaphoreType.DMA,
        pltpu.SemaphoreType.DMA,
        pltpu.SemaphoreType.DMA,
        pltpu.SemaphoreType.DMA,
    ],
)
def _sc_prop(src_hbm, dst_hbm, vals_hbm, zeros_hbm, out_hbm,
             src_v, dst_v, rows_v, acc_sh, gs0, gs1, ss0, ss1):
    c = lax.axis_index("c")
    s = lax.axis_index("s")
    wid = s * NC + c
    gsem = (gs0, gs1)
    ssem = (ss0, ss1)

    def gather(i, half):
        pltpu.async_copy(vals_hbm.at[src_v.at[pl.ds(i * W, W)]],
                         rows_v.at[pl.ds(half * W, W)], gsem[half])

    def scatter(i, half):
        pltpu.async_copy(rows_v.at[pl.ds(half * W, W)],
                         acc_sh.at[dst_v.at[i]], ssem[half], add=True)

    def drain(sems, half):
        pltpu.make_async_copy(vals_hbm.at[pl.ds(0, W)],
                              rows_v.at[pl.ds(half * W, W)],
                              sems[half]).wait()

    pltpu.sync_copy(src_hbm.at[wid], src_v)
    pltpu.sync_copy(dst_hbm.at[wid], dst_v)
    gather(0, 0)
    _part_init(zeros_hbm, acc_sh, s)
    plsc.subcore_barrier()

    @pl.loop(0, NCHUNK // 2)
    def _(k):
        i0 = 2 * k
        drain(gsem, 0)
        gather(i0 + 1, 1)
        scatter(i0, 0)
        drain(ssem, 0)
        drain(gsem, 1)

        @pl.when(k < NCHUNK // 2 - 1)
        def _():
            gather(i0 + 2, 0)

        scatter(i0 + 1, 1)
        drain(ssem, 1)

    plsc.subcore_barrier()
    _part_copyout(acc_sh, out_hbm, c, s)


def _tc1_body(degp_ref, x_ref, w1_ref, xw1p_ref, dis_ref):
    deg = jnp.sum(degp_ref[...], axis=0)[0:N, None] + 1.0
    dis = 1.0 / jnp.sqrt(deg)
    xw1 = jnp.dot(x_ref[...], w1_ref[...], preferred_element_type=jnp.float32)
    xw1p_ref[0:N, :] = dis * xw1
    xw1p_ref[N:NP, :] = jnp.zeros((PADR, F), jnp.float32)
    dis_ref[...] = dis


def _tc2_body(acc_ref, xw1p_ref, dis_ref, b1_ref, hp_ref):
    t = (dis_ref[...] * (acc_ref[0] + acc_ref[1] + xw1p_ref[0:N, :])
         + b1_ref[...])
    hp_ref[0:N, :] = dis_ref[...] * jnp.maximum(t, 0.0)
    hp_ref[N:NP, :] = jnp.zeros((PADR, F), jnp.float32)


def _tc3_body(acc_ref, hp_ref, dis_ref, w2_ref, b2_ref, out_ref):
    g = dis_ref[...] * (acc_ref[0] + acc_ref[1] + hp_ref[0:N, :])
    out_ref[...] = (jnp.dot(g, w2_ref[...],
                            preferred_element_type=jnp.float32) + b2_ref[...])


def kernel(x, edge_index, W1, b1, W2, b2):
    x = x.astype(jnp.float32)
    ei = edge_index.astype(jnp.int32)
    padidx = N + (jnp.arange(PAD_E, dtype=jnp.int32) % PADR)
    srcp = jnp.concatenate([ei[0], padidx])
    dstp = jnp.concatenate([ei[1], padidx])
    src2 = srcp.reshape(NW, EPTP)
    dst3 = dstp.reshape(NW, NCHUNK, W)
    zeros_f = jnp.zeros((B0, F), jnp.float32)

    degp = _sc_degree(dst3)

    xw1p, dis = pl.pallas_call(
        _tc1_body,
        out_shape=[
            jax.ShapeDtypeStruct((NP, F), jnp.float32),
            jax.ShapeDtypeStruct((N, 1), jnp.float32),
        ],
    )(degp, x, W1)

    acc1 = _sc_prop(src2, dst3, xw1p, zeros_f)

    hp = pl.pallas_call(
        _tc2_body,
        out_shape=jax.ShapeDtypeStruct((NP, F), jnp.float32),
    )(acc1, xw1p, dis, b1)

    acc2 = _sc_prop(src2, dst3, hp, zeros_f)

    out = pl.pallas_call(
        _tc3_body,
        out_shape=jax.ShapeDtypeStruct((N, C), jnp.float32),
    )(acc2, hp, dis, W2, b2)

    return out


# speedup vs baseline: 29.3095x; 1.1743x over previous
import dataclasses
import functools

import jax
import jax.numpy as jnp
from jax import lax
from jax.experimental import pallas as pl
from jax.experimental.pallas import tpu as pltpu
from jax.experimental.pallas import tpu_sc as plsc

N = 10000
F = 128
C = 16
E = 320000
NC = 2
NS = 16
NW = NC * NS
W = 128
NCHUNK = 80
EPTP = NCHUNK * W
PADR = 64
NP = N + PADR
PAD_E = NW * EPTP - E

B0 = 632
BL = N - (NS - 1) * B0

_mesh = plsc.VectorSubcoreMesh(core_axis_name="c", subcore_axis_name="s")

_cp = pltpu.CompilerParams()
if "needs_layout_passes" in pltpu.CompilerParams.__dataclass_fields__:
    _cp = dataclasses.replace(_cp, needs_layout_passes=False)


def _part_init(zeros_hbm, acc_sh, s):
    base = pl.multiple_of(s * B0, 8)

    @pl.when(s < NS - 1)
    def _():
        pltpu.sync_copy(zeros_hbm, acc_sh.at[pl.ds(base, B0)])

    @pl.when(s == NS - 1)
    def _():
        pltpu.sync_copy(zeros_hbm.at[pl.ds(0, BL)], acc_sh.at[pl.ds(base, BL)])


def _part_copyout(acc_sh, out_hbm, c, s):
    base = pl.multiple_of(s * B0, 8)

    @pl.when(s < NS - 1)
    def _():
        pltpu.sync_copy(acc_sh.at[pl.ds(base, B0)],
                        out_hbm.at[c, pl.ds(base, B0)])

    @pl.when(s == NS - 1)
    def _():
        pltpu.sync_copy(acc_sh.at[pl.ds(base, BL)],
                        out_hbm.at[c, pl.ds(base, BL)])


@functools.partial(
    pl.kernel, mesh=_mesh,
    out_type=jax.ShapeDtypeStruct((NW, NP), jnp.float32),
    compiler_params=_cp,
    scratch_types=[
        pltpu.VMEM((NCHUNK, W), jnp.int32),
        pltpu.VMEM((NP,), jnp.float32),
    ],
)
def _sc_degree(dst_hbm, out_hbm, dst_v, hist_v):
    c = lax.axis_index("c")
    s = lax.axis_index("s")
    wid = s * NC + c
    pltpu.sync_copy(dst_hbm.at[wid], dst_v)

    @pl.loop(0, NP // 16)
    def _(i):
        hist_v[pl.ds(i * 16, 16)] = jnp.zeros((16,), jnp.float32)

    ones16 = jnp.ones((16,), jnp.float32)

    @pl.loop(0, NCHUNK)
    def _(i):
        for j in range(W // 16):
            idx16 = dst_v[i, pl.ds(j * 16, 16)]
            plsc.addupdate_scatter(hist_v, [idx16], ones16)

    pltpu.sync_copy(hist_v, out_hbm.at[wid])


NSLOT = 4


@functools.partial(
    pl.kernel, mesh=_mesh,
    out_type=jax.ShapeDtypeStruct((NC, N, F), jnp.float32),
    scratch_types=[
        pltpu.VMEM((NSLOT, W), jnp.int32),
        pltpu.VMEM((NCHUNK, W), jnp.int32),
        pltpu.VMEM((2 * W, F), jnp.float32),
        pltpu.VMEM_SHARED((NP, F), jnp.float32),
        pltpu.SemaphoreType.DMA,
        pltpu.SemaphoreType.DMA,
        pltpu.SemaphoreType.DMA,
        pltpu.SemaphoreType.DMA,
        pltpu.SemaphoreType.DMA,
        pltpu.SemaphoreType.DMA,
        pltpu.SemaphoreType.DMA,
        pltpu.SemaphoreType.DMA,
    ],
)
def _sc_prop(src_hbm, dst_hbm, vals_hbm, zeros_hbm, out_hbm,
             src_v, dst_v, rows_v, acc_sh,
             gs0, gs1, ss0, ss1, is0, is1, is2, is3):
    c = lax.axis_index("c")
    s = lax.axis_index("s")
    wid = s * NC + c
    gsem = (gs0, gs1)
    ssem = (ss0, ss1)
    isem = (is0, is1, is2, is3)

    def load_idx(i, sl):
        pltpu.async_copy(src_hbm.at[wid, i], src_v.at[sl], isem[sl])

    def gather(i, half, sl):
        pltpu.async_copy(vals_hbm.at[src_v.at[sl]],
                         rows_v.at[pl.ds(half * W, W)], gsem[half])

    def scatter(i, half):
        pltpu.async_copy(rows_v.at[pl.ds(half * W, W)],
                         acc_sh.at[dst_v.at[i]], ssem[half], add=True)

    def drain_rows(sems, half):
        pltpu.make_async_copy(vals_hbm.at[pl.ds(0, W)],
                              rows_v.at[pl.ds(half * W, W)],
                              sems[half]).wait()

    def drain_idx(sl):
        pltpu.make_async_copy(dst_hbm.at[0, 0], src_v.at[sl],
                              isem[sl]).wait()

    pltpu.sync_copy(dst_hbm.at[wid], dst_v)
    for sl in range(NSLOT):
        load_idx(sl, sl)
    drain_idx(0)
    gather(0, 0, 0)
    _part_init(zeros_hbm, acc_sh, s)
    plsc.subcore_barrier()

    @pl.loop(0, NCHUNK // NSLOT)
    def _(k):
        for q in range(NSLOT):
            half = q % 2
            i = NSLOT * k + q
            drain_rows(gsem, half)

            @pl.when(i + NSLOT < NCHUNK)
            def _():
                load_idx(i + NSLOT, q)

            @pl.when(i + 1 < NCHUNK)
            def _():
                drain_idx((q + 1) % NSLOT)
                gather(i + 1, 1 - half, (q + 1) % NSLOT)

            scatter(i, half)
            drain_rows(ssem, half)

    plsc.subcore_barrier()
    _part_copyout(acc_sh, out_hbm, c, s)


def _tc1_body(degp_ref, x_ref, w1_ref, xw1p_ref, dis_ref):
    deg = jnp.sum(degp_ref[...], axis=0)[0:N, None] + 1.0
    dis = 1.0 / jnp.sqrt(deg)
    xw1 = jnp.dot(x_ref[...], w1_ref[...], preferred_element_type=jnp.float32)
    xw1p_ref[0:N, :] = dis * xw1
    xw1p_ref[N:NP, :] = jnp.zeros((PADR, F), jnp.float32)
    dis_ref[...] = dis


def _tc2_body(acc_ref, xw1p_ref, dis_ref, b1_ref, hp_ref):
    t = (dis_ref[...] * (acc_ref[0] + acc_ref[1] + xw1p_ref[0:N, :])
         + b1_ref[...])
    hp_ref[0:N, :] = dis_ref[...] * jnp.maximum(t, 0.0)
    hp_ref[N:NP, :] = jnp.zeros((PADR, F), jnp.float32)


def _tc3_body(acc_ref, hp_ref, dis_ref, w2_ref, b2_ref, out_ref):
    g = dis_ref[...] * (acc_ref[0] + acc_ref[1] + hp_ref[0:N, :])
    out_ref[...] = (jnp.dot(g, w2_ref[...],
                            preferred_element_type=jnp.float32) + b2_ref[...])


def kernel(x, edge_index, W1, b1, W2, b2):
    x = x.astype(jnp.float32)
    ei = edge_index.astype(jnp.int32)
    padidx = N + (jnp.arange(PAD_E, dtype=jnp.int32) % PADR)
    srcp = jnp.concatenate([ei[0], padidx])
    dstp = jnp.concatenate([ei[1], padidx])
    src3 = srcp.reshape(NW, NCHUNK, W)
    dst3 = dstp.reshape(NW, NCHUNK, W)
    zeros_f = jnp.zeros((B0, F), jnp.float32)

    degp = _sc_degree(dst3)

    xw1p, dis = pl.pallas_call(
        _tc1_body,
        out_shape=[
            jax.ShapeDtypeStruct((NP, F), jnp.float32),
            jax.ShapeDtypeStruct((N, 1), jnp.float32),
        ],
    )(degp, x, W1)

    acc1 = _sc_prop(src3, dst3, xw1p, zeros_f)

    hp = pl.pallas_call(
        _tc2_body,
        out_shape=jax.ShapeDtypeStruct((NP, F), jnp.float32),
    )(acc1, xw1p, dis, b1)

    acc2 = _sc_prop(src3, dst3, hp, zeros_f)

    out = pl.pallas_call(
        _tc3_body,
        out_shape=jax.ShapeDtypeStruct((N, C), jnp.float32),
    )(acc2, hp, dis, W2, b2)

    return out


# speedup vs baseline: 33.9801x; 1.1594x over previous
import dataclasses
import functools

import jax
import jax.numpy as jnp
from jax import lax
from jax.experimental import pallas as pl
from jax.experimental.pallas import tpu as pltpu
from jax.experimental.pallas import tpu_sc as plsc

N = 10000
F = 128
C = 16
E = 320000
NC = 2
NS = 16
NW = NC * NS
W = 128
NCHUNK = 80
EPTP = NCHUNK * W
PADR = 64
NP = N + PADR
PAD_E = NW * EPTP - E

B0 = 632
BL = N - (NS - 1) * B0

_mesh = plsc.VectorSubcoreMesh(core_axis_name="c", subcore_axis_name="s")

_cp = pltpu.CompilerParams()
if "needs_layout_passes" in pltpu.CompilerParams.__dataclass_fields__:
    _cp = dataclasses.replace(_cp, needs_layout_passes=False)


def _part_init(zeros_hbm, acc_sh, s):
    base = pl.multiple_of(s * B0, 8)

    @pl.when(s < NS - 1)
    def _():
        pltpu.sync_copy(zeros_hbm, acc_sh.at[pl.ds(base, B0)])

    @pl.when(s == NS - 1)
    def _():
        pltpu.sync_copy(zeros_hbm.at[pl.ds(0, BL)], acc_sh.at[pl.ds(base, BL)])


def _part_copyout(acc_sh, out_hbm, c, s):
    base = pl.multiple_of(s * B0, 8)

    @pl.when(s < NS - 1)
    def _():
        pltpu.sync_copy(acc_sh.at[pl.ds(base, B0)],
                        out_hbm.at[c, pl.ds(base, B0)])

    @pl.when(s == NS - 1)
    def _():
        pltpu.sync_copy(acc_sh.at[pl.ds(base, BL)],
                        out_hbm.at[c, pl.ds(base, BL)])


@functools.partial(
    pl.kernel, mesh=_mesh,
    out_type=jax.ShapeDtypeStruct((NW, NP), jnp.float32),
    compiler_params=_cp,
    scratch_types=[
        pltpu.VMEM((NCHUNK, W), jnp.int32),
        pltpu.VMEM((NP,), jnp.float32),
    ],
)
def _sc_degree(dst_hbm, out_hbm, dst_v, hist_v):
    c = lax.axis_index("c")
    s = lax.axis_index("s")
    wid = s * NC + c
    pltpu.sync_copy(dst_hbm.at[wid], dst_v)

    @pl.loop(0, NP // 16)
    def _(i):
        hist_v[pl.ds(i * 16, 16)] = jnp.zeros((16,), jnp.float32)

    ones16 = jnp.ones((16,), jnp.float32)

    @pl.loop(0, NCHUNK)
    def _(i):
        for j in range(W // 16):
            idx16 = dst_v[i, pl.ds(j * 16, 16)]
            plsc.addupdate_scatter(hist_v, [idx16], ones16)

    pltpu.sync_copy(hist_v, out_hbm.at[wid])


NSLOT = 4


@functools.partial(
    pl.kernel, mesh=_mesh,
    out_type=jax.ShapeDtypeStruct((NC, N, F), jnp.float32),
    scratch_types=[
        pltpu.VMEM((NSLOT, W), jnp.int32),
        pltpu.VMEM((NCHUNK, W), jnp.int32),
        pltpu.VMEM((2 * W, F), jnp.float32),
        pltpu.VMEM_SHARED((NP, F), jnp.float32),
        pltpu.SemaphoreType.DMA,
        pltpu.SemaphoreType.DMA,
        pltpu.SemaphoreType.DMA,
        pltpu.SemaphoreType.DMA,
        pltpu.SemaphoreType.DMA,
        pltpu.SemaphoreType.DMA,
        pltpu.SemaphoreType.DMA,
        pltpu.SemaphoreType.DMA,
    ],
)
def _sc_prop(src_hbm, dst_hbm, vals_hbm, zeros_hbm, out_hbm,
             src_v, dst_v, rows_v, acc_sh,
             gs0, gs1, ss0, ss1, is0, is1, is2, is3):
    c = lax.axis_index("c")
    s = lax.axis_index("s")
    wid = s * NC + c
    gsem = (gs0, gs1)
    ssem = (ss0, ss1)
    isem = (is0, is1, is2, is3)

    def load_idx(i, sl):
        pltpu.async_copy(src_hbm.at[wid, i], src_v.at[sl], isem[sl])

    def gather(i, half, sl):
        pltpu.async_copy(vals_hbm.at[src_v.at[sl]],
                         rows_v.at[pl.ds(half * W, W)], gsem[half])

    def scatter(i, half):
        pltpu.async_copy(rows_v.at[pl.ds(half * W, W)],
                         acc_sh.at[dst_v.at[i]], ssem[half], add=True)

    def drain_rows(sems, half):
        pltpu.make_async_copy(vals_hbm.at[pl.ds(0, W)],
                              rows_v.at[pl.ds(half * W, W)],
                              sems[half]).wait()

    def drain_idx(sl):
        pltpu.make_async_copy(dst_hbm.at[0, 0], src_v.at[sl],
                              isem[sl]).wait()

    pltpu.sync_copy(dst_hbm.at[wid], dst_v)
    for sl in range(NSLOT):
        load_idx(sl, sl)
    drain_idx(0)
    gather(0, 0, 0)
    _part_init(zeros_hbm, acc_sh, s)
    plsc.subcore_barrier()

    @pl.loop(0, NCHUNK // NSLOT)
    def _(k):
        for q in range(NSLOT):
            half = q % 2
            i = NSLOT * k + q

            @pl.when(i + 1 < NCHUNK)
            def _():
                drain_idx((q + 1) % NSLOT)
                gather(i + 1, 1 - half, (q + 1) % NSLOT)

            drain_rows(gsem, half)

            @pl.when(i + NSLOT < NCHUNK)
            def _():
                load_idx(i + NSLOT, q)

            scatter(i, half)
            drain_rows(ssem, half)

    plsc.subcore_barrier()
    _part_copyout(acc_sh, out_hbm, c, s)


def _tc1_body(degp_ref, x_ref, w1_ref, xw1p_ref, dis_ref):
    deg = jnp.sum(degp_ref[...], axis=0)[0:N, None] + 1.0
    dis = 1.0 / jnp.sqrt(deg)
    xw1 = jnp.dot(x_ref[...], w1_ref[...], preferred_element_type=jnp.float32)
    xw1p_ref[0:N, :] = dis * xw1
    xw1p_ref[N:NP, :] = jnp.zeros((PADR, F), jnp.float32)
    dis_ref[...] = dis


def _tc2_body(acc_ref, xw1p_ref, dis_ref, b1_ref, hp_ref):
    t = (dis_ref[...] * (acc_ref[0] + acc_ref[1] + xw1p_ref[0:N, :])
         + b1_ref[...])
    hp_ref[0:N, :] = dis_ref[...] * jnp.maximum(t, 0.0)
    hp_ref[N:NP, :] = jnp.zeros((PADR, F), jnp.float32)


def _tc3_body(acc_ref, hp_ref, dis_ref, w2_ref, b2_ref, out_ref):
    g = dis_ref[...] * (acc_ref[0] + acc_ref[1] + hp_ref[0:N, :])
    out_ref[...] = (jnp.dot(g, w2_ref[...],
                            preferred_element_type=jnp.float32) + b2_ref[...])


def kernel(x, edge_index, W1, b1, W2, b2):
    x = x.astype(jnp.float32)
    ei = edge_index.astype(jnp.int32)
    padidx = N + (jnp.arange(PAD_E, dtype=jnp.int32) % PADR)
    srcp = jnp.concatenate([ei[0], padidx])
    dstp = jnp.concatenate([ei[1], padidx])
    src3 = srcp.reshape(NW, NCHUNK, W)
    dst3 = dstp.reshape(NW, NCHUNK, W)
    zeros_f = jnp.zeros((B0, F), jnp.float32)

    degp = _sc_degree(dst3)

    xw1p, dis = pl.pallas_call(
        _tc1_body,
        out_shape=[
            jax.ShapeDtypeStruct((NP, F), jnp.float32),
            jax.ShapeDtypeStruct((N, 1), jnp.float32),
        ],
    )(degp, x, W1)

    acc1 = _sc_prop(src3, dst3, xw1p, zeros_f)

    hp = pl.pallas_call(
        _tc2_body,
        out_shape=jax.ShapeDtypeStruct((NP, F), jnp.float32),
    )(acc1, xw1p, dis, b1)

    acc2 = _sc_prop(src3, dst3, hp, zeros_f)

    out = pl.pallas_call(
        _tc3_body,
        out_shape=jax.ShapeDtypeStruct((N, C), jnp.float32),
    )(acc2, hp, dis, W2, b2)

    return out


# speedup vs baseline: 36.0209x; 1.0601x over previous
import dataclasses
import functools

import jax
import jax.numpy as jnp
from jax import lax
from jax.experimental import pallas as pl
from jax.experimental.pallas import tpu as pltpu
from jax.experimental.pallas import tpu_sc as plsc

N = 10000
F = 128
C = 16
E = 320000
NC = 2
NS = 16
NW = NC * NS
W = 128
GCH = E // W
TSTEPS = (GCH + NW - 1) // NW
NSLOT = 4
TLOOP = ((TSTEPS + NSLOT - 1) // NSLOT) * NSLOT
FAKE0 = NW * TLOOP * W - E

B0 = 632
BL = N - (NS - 1) * B0

_mesh = plsc.VectorSubcoreMesh(core_axis_name="c", subcore_axis_name="s")

_cp = pltpu.CompilerParams()
if "needs_layout_passes" in pltpu.CompilerParams.__dataclass_fields__:
    _cp = dataclasses.replace(_cp, needs_layout_passes=False)


def _part_init(zeros_hbm, acc_sh, s):
    base = pl.multiple_of(s * B0, 8)

    @pl.when(s < NS - 1)
    def _():
        pltpu.sync_copy(zeros_hbm, acc_sh.at[pl.ds(base, B0)])

    @pl.when(s == NS - 1)
    def _():
        pltpu.sync_copy(zeros_hbm.at[pl.ds(0, BL)], acc_sh.at[pl.ds(base, BL)])


def _part_copyout(acc_sh, out_hbm, c, s):
    base = pl.multiple_of(s * B0, 8)

    @pl.when(s < NS - 1)
    def _():
        pltpu.sync_copy(acc_sh.at[pl.ds(base, B0)],
                        out_hbm.at[c, pl.ds(base, B0)])

    @pl.when(s == NS - 1)
    def _():
        pltpu.sync_copy(acc_sh.at[pl.ds(base, BL)],
                        out_hbm.at[c, pl.ds(base, BL)])


@functools.partial(
    pl.kernel, mesh=_mesh,
    out_type=jax.ShapeDtypeStruct((NW, N), jnp.float32),
    compiler_params=_cp,
    scratch_types=[
        pltpu.VMEM((TLOOP, W), jnp.int32),
        pltpu.VMEM((N,), jnp.float32),
        pltpu.SemaphoreType.DMA,
    ],
)
def _sc_degree(ei_hbm, out_hbm, stage_v, hist_v, dsem):
    c = lax.axis_index("c")
    s = lax.axis_index("s")
    wid = s * NC + c

    zeros16 = jnp.zeros((16,), jnp.int32)
    for r in (TLOOP - 2, TLOOP - 1):
        for j in range(W // 16):
            stage_v[r, pl.ds(j * 16, 16)] = zeros16

    @pl.loop(0, TLOOP)
    def _(t):
        g = t * NW + wid

        @pl.when(g < GCH)
        def _():
            off = pl.multiple_of(g * W, 128)
            pltpu.async_copy(ei_hbm.at[1, pl.ds(off, W)], stage_v.at[t], dsem)

    @pl.loop(0, N // 16)
    def _(i):
        hist_v[pl.ds(i * 16, 16)] = jnp.zeros((16,), jnp.float32)

    @pl.loop(0, TLOOP)
    def _(t):
        @pl.when(t * NW + wid < GCH)
        def _():
            pltpu.make_async_copy(ei_hbm.at[1, pl.ds(0, W)],
                                  stage_v.at[t], dsem).wait()

    ones16 = jnp.ones((16,), jnp.float32)

    @pl.loop(0, TLOOP)
    def _(t):
        for j in range(W // 16):
            idx16 = stage_v[t, pl.ds(j * 16, 16)]
            plsc.addupdate_scatter(hist_v, [idx16], ones16)

    pltpu.sync_copy(hist_v, out_hbm.at[wid])


@functools.partial(
    pl.kernel, mesh=_mesh,
    out_type=jax.ShapeDtypeStruct((NC, N, F), jnp.float32),
    scratch_types=[
        pltpu.VMEM((NSLOT, W), jnp.int32),
        pltpu.VMEM((NSLOT, W), jnp.int32),
        pltpu.VMEM((2 * W, F), jnp.float32),
        pltpu.VMEM_SHARED((N, F), jnp.float32),
        pltpu.SemaphoreType.DMA,
        pltpu.SemaphoreType.DMA,
        pltpu.SemaphoreType.DMA,
        pltpu.SemaphoreType.DMA,
        pltpu.SemaphoreType.DMA,
        pltpu.SemaphoreType.DMA,
        pltpu.SemaphoreType.DMA,
        pltpu.SemaphoreType.DMA,
        pltpu.SemaphoreType.DMA,
        pltpu.SemaphoreType.DMA,
        pltpu.SemaphoreType.DMA,
        pltpu.SemaphoreType.DMA,
    ],
)
def _sc_prop(ei_hbm, vals_hbm, zeros_hbm, out_hbm,
             sidx_v, didx_v, rows_v, acc_sh,
             gs0, gs1, ss0, ss1, si0, si1, si2, si3, di0, di1, di2, di3):
    c = lax.axis_index("c")
    s = lax.axis_index("s")
    wid = s * NC + c
    gsem = (gs0, gs1)
    ssem = (ss0, ss1)
    sisem = (si0, si1, si2, si3)
    disem = (di0, di1, di2, di3)

    def load_idx(row, t, q, slots, sems):
        off = pl.multiple_of((t * NW + wid) * W, 128)
        pltpu.async_copy(ei_hbm.at[row, pl.ds(off, W)], slots.at[q], sems[q])

    def gather(q, half):
        pltpu.async_copy(vals_hbm.at[sidx_v.at[q]],
                         rows_v.at[pl.ds(half * W, W)], gsem[half])

    def scatter(q, half):
        pltpu.async_copy(rows_v.at[pl.ds(half * W, W)],
                         acc_sh.at[didx_v.at[q]], ssem[half], add=True)

    def drain_rows(sems, half):
        pltpu.make_async_copy(vals_hbm.at[pl.ds(0, W)],
                              rows_v.at[pl.ds(half * W, W)],
                              sems[half]).wait()

    def drain_slot(slots, sems, q):
        pltpu.make_async_copy(ei_hbm.at[0, pl.ds(0, W)], slots.at[q],
                              sems[q]).wait()

    for q in range(NSLOT):
        load_idx(0, q, q, sidx_v, sisem)
        load_idx(1, q, q, didx_v, disem)
    drain_slot(sidx_v, sisem, 0)
    gather(0, 0)
    _part_init(zeros_hbm, acc_sh, s)
    plsc.subcore_barrier()

    @pl.loop(0, TLOOP // NSLOT)
    def _(k):
        for q in range(NSLOT):
            half = q % 2
            t = NSLOT * k + q
            g = t * NW + wid

            @pl.when(g + NW < GCH)
            def _():
                drain_slot(sidx_v, sisem, (q + 1) % NSLOT)
                gather((q + 1) % NSLOT, 1 - half)

            @pl.when(g < GCH)
            def _():
                drain_rows(gsem, half)

            @pl.when(g + NSLOT * NW < GCH)
            def _():
                load_idx(0, t + NSLOT, q, sidx_v, sisem)

            @pl.when(g < GCH)
            def _():
                drain_slot(didx_v, disem, q)
                scatter(q, half)
                drain_rows(ssem, half)

            @pl.when(g + NSLOT * NW < GCH)
            def _():
                load_idx(1, t + NSLOT, q, didx_v, disem)

    plsc.subcore_barrier()
    _part_copyout(acc_sh, out_hbm, c, s)


def _tc1_body(degp_ref, x_ref, w1_ref, xw1p_ref, dis_ref):
    dsum = jnp.sum(degp_ref[...], axis=0)[:, None]
    row = lax.broadcasted_iota(jnp.int32, (N, 1), 0)
    deg = dsum - jnp.where(row == 0, jnp.float32(FAKE0), 0.0) + 1.0
    dis = 1.0 / jnp.sqrt(deg)
    xw1 = jnp.dot(x_ref[...], w1_ref[...], preferred_element_type=jnp.float32)
    xw1p_ref[...] = dis * xw1
    dis_ref[...] = dis


def _tc2_body(acc_ref, xw1p_ref, dis_ref, b1_ref, hp_ref):
    t = dis_ref[...] * (acc_ref[0] + acc_ref[1] + xw1p_ref[...]) + b1_ref[...]
    hp_ref[...] = dis_ref[...] * jnp.maximum(t, 0.0)


def _tc3_body(acc_ref, hp_ref, dis_ref, w2_ref, b2_ref, out_ref):
    g = dis_ref[...] * (acc_ref[0] + acc_ref[1] + hp_ref[...])
    out_ref[...] = (jnp.dot(g, w2_ref[...],
                            preferred_element_type=jnp.float32) + b2_ref[...])


def kernel(x, edge_index, W1, b1, W2, b2):
    x = x.astype(jnp.float32)
    ei = edge_index.astype(jnp.int32)
    zeros_f = jnp.zeros((B0, F), jnp.float32)

    degp = _sc_degree(ei)

    xw1p, dis = pl.pallas_call(
        _tc1_body,
        out_shape=[
            jax.ShapeDtypeStruct((N, F), jnp.float32),
            jax.ShapeDtypeStruct((N, 1), jnp.float32),
        ],
    )(degp, x, W1)

    acc1 = _sc_prop(ei, xw1p, zeros_f)

    hp = pl.pallas_call(
        _tc2_body,
        out_shape=jax.ShapeDtypeStruct((N, F), jnp.float32),
    )(acc1, xw1p, dis, b1)

    acc2 = _sc_prop(ei, hp, zeros_f)

    out = pl.pallas_call(
        _tc3_body,
        out_shape=jax.ShapeDtypeStruct((N, C), jnp.float32),
    )(acc2, hp, dis, W2, b2)

    return out
